# baseline (speedup 1.0000x reference)
import jax
import jax.numpy as jnp
from jax.experimental import pallas as pl
from jax.experimental.pallas import tpu as pltpu

_NE = 1024
_D = 64
_P = 1024


def _body(x_ref, e_ref, o_ref, e2_ref):
    e = e_ref[...]

    @pl.when((pl.program_id(0) == 0) & (pl.program_id(1) == 0))
    def _init():
        e2_ref[0, :] = jnp.sum(e * e, axis=1)

    ii = jax.lax.broadcasted_iota(jnp.int32, (_D, _D), 0)
    jj = jax.lax.broadcasted_iota(jnp.int32, (_D, _D), 1)
    eye = (ii == jj).astype(jnp.float32)
    x = jax.lax.dot_general(x_ref[0], eye, (((0,), (0,)), ((), ())),
                            precision=jax.lax.Precision.HIGHEST)
    x2 = jnp.sum(x * x, axis=1, keepdims=True)
    e2 = e2_ref[...]
    mm2 = jax.lax.dot_general(x, e + e, (((1,), (1,)), ((), ())))
    dist = (x2 + e2) - mm2
    m = jnp.min(dist, axis=1, keepdims=True)
    jidx = jax.lax.broadcasted_iota(jnp.int32, (_P, _NE), 1).astype(jnp.float32)
    idx = jnp.min(jnp.where(dist == m, jidx, float(_NE)), axis=1,
                  keepdims=True)
    oh = (jidx == idx).astype(jnp.float32)
    q = jax.lax.dot_general(oh, e, (((1,), (0,)), ((), ())))
    o_ref[0] = jnp.transpose(x + (q - x), (1, 0))


def kernel(inputs, embedding):
    b, c, h, w = inputs.shape
    xf = inputs.reshape(b, c, h * w)
    npix = h * w
    out = pl.pallas_call(
        _body,
        grid=(b, npix // _P),
        in_specs=[pl.BlockSpec((1, c, _P), lambda i, j: (i, 0, j)),
                  pl.BlockSpec((_NE, _D), lambda i, j: (0, 0))],
        out_specs=pl.BlockSpec((1, c, _P), lambda i, j: (i, 0, j)),
        out_shape=jax.ShapeDtypeStruct((b, c, npix), jnp.float32),
        scratch_shapes=[pltpu.VMEM((1, _NE), jnp.float32)],
    )(xf, embedding)
    return out.reshape(b, c, h, w)


# speedup vs baseline: 1.1145x; 1.1145x over previous
import jax
import jax.numpy as jnp
from jax.experimental import pallas as pl
from jax.experimental.pallas import tpu as pltpu

_NE = 1024
_D = 64
_P = 1024


def _body(x_ref, e_ref, o_ref, e2_ref):
    e = e_ref[...]

    @pl.when((pl.program_id(0) == 0) & (pl.program_id(1) == 0))
    def _init():
        e2_ref[0, :] = jnp.sum(e * e, axis=1)

    x = jnp.transpose(x_ref[0], (1, 0))
    x2 = jnp.sum(x * x, axis=1, keepdims=True)
    e2 = e2_ref[...]
    mm2 = jax.lax.dot_general(x, e + e, (((1,), (1,)), ((), ())))
    dist = (x2 + e2) - mm2
    m = jnp.min(dist, axis=1, keepdims=True)
    jidx = jax.lax.broadcasted_iota(jnp.int32, (_P, _NE), 1).astype(jnp.float32)
    idx = jnp.min(jnp.where(dist == m, jidx, float(_NE)), axis=1,
                  keepdims=True)
    oh = (jidx == idx).astype(jnp.float32)
    q = jax.lax.dot_general(oh, e, (((1,), (0,)), ((), ())))
    o_ref[0] = jnp.transpose(x + (q - x), (1, 0))


def kernel(inputs, embedding):
    b, c, h, w = inputs.shape
    xf = inputs.reshape(b, c, h * w)
    npix = h * w
    out = pl.pallas_call(
        _body,
        grid=(b, npix // _P),
        in_specs=[pl.BlockSpec((1, c, _P), lambda i, j: (i, 0, j)),
                  pl.BlockSpec((_NE, _D), lambda i, j: (0, 0))],
        out_specs=pl.BlockSpec((1, c, _P), lambda i, j: (i, 0, j)),
        out_shape=jax.ShapeDtypeStruct((b, c, npix), jnp.float32),
        scratch_shapes=[pltpu.VMEM((1, _NE), jnp.float32)],
    )(xf, embedding)
    return out.reshape(b, c, h, w)


# speedup vs baseline: 1.5566x; 1.3967x over previous
import jax
import jax.numpy as jnp
from jax.experimental import pallas as pl
from jax.experimental.pallas import tpu as pltpu

_NE = 1024
_D = 64
_P = 1024


def _body(x_ref, e_ref, o_ref, e2_ref):
    e = e_ref[...]

    @pl.when((pl.program_id(0) == 0) & (pl.program_id(1) == 0))
    def _init():
        e2_ref[...] = jnp.sum(e * e, axis=1, keepdims=True)

    x = x_ref[0]
    x2 = jnp.sum(x * x, axis=0, keepdims=True)
    e2 = e2_ref[...]
    mm2 = jax.lax.dot_general(e + e, x, (((1,), (0,)), ((), ())))
    dist = (x2 + e2) - mm2
    m = jnp.min(dist, axis=0, keepdims=True)
    jidx = jax.lax.broadcasted_iota(jnp.int32, (_NE, _P), 0).astype(jnp.float32)
    idx = jnp.min(jnp.where(dist == m, jidx, float(_NE)), axis=0,
                  keepdims=True)
    oh = (jidx == idx).astype(jnp.float32)
    q = jax.lax.dot_general(e, oh, (((0,), (0,)), ((), ())))
    o_ref[0] = x + (q - x)


def kernel(inputs, embedding):
    b, c, h, w = inputs.shape
    xf = inputs.reshape(b, c, h * w)
    npix = h * w
    out = pl.pallas_call(
        _body,
        grid=(b, npix // _P),
        in_specs=[pl.BlockSpec((1, c, _P), lambda i, j: (i, 0, j)),
                  pl.BlockSpec((_NE, _D), lambda i, j: (0, 0))],
        out_specs=pl.BlockSpec((1, c, _P), lambda i, j: (i, 0, j)),
        out_shape=jax.ShapeDtypeStruct((b, c, npix), jnp.float32),
        scratch_shapes=[pltpu.VMEM((_NE, 1), jnp.float32)],
    )(xf, embedding)
    return out.reshape(b, c, h, w)


# speedup vs baseline: 1.7006x; 1.0925x over previous
import jax
import jax.numpy as jnp
from jax.experimental import pallas as pl
from jax.experimental.pallas import tpu as pltpu

_NE = 1024
_D = 64
_P = 2048


def _body(x_ref, e_ref, o_ref, e2_ref):
    e = e_ref[...]

    @pl.when((pl.program_id(0) == 0) & (pl.program_id(1) == 0))
    def _init():
        e2_ref[...] = jnp.sum(e * e, axis=1, keepdims=True)

    x = x_ref[0]
    x2 = jnp.sum(x * x, axis=0, keepdims=True)
    e2 = e2_ref[...]
    mm2 = jax.lax.dot_general(e + e, x, (((1,), (0,)), ((), ())))
    dist = (x2 + e2) - mm2
    m = jnp.min(dist, axis=0, keepdims=True)
    jidx = jax.lax.broadcasted_iota(jnp.int32, (_NE, _P), 0).astype(jnp.float32)
    idx = jnp.min(jnp.where(dist == m, jidx, float(_NE)), axis=0,
                  keepdims=True)
    oh = (jidx == idx).astype(jnp.float32)
    q = jax.lax.dot_general(e, oh, (((0,), (0,)), ((), ())))
    o_ref[0] = x + (q - x)


def kernel(inputs, embedding):
    b, c, h, w = inputs.shape
    xf = inputs.reshape(b, c, h * w)
    npix = h * w
    out = pl.pallas_call(
        _body,
        grid=(b, npix // _P),
        in_specs=[pl.BlockSpec((1, c, _P), lambda i, j: (i, 0, j)),
                  pl.BlockSpec((_NE, _D), lambda i, j: (0, 0))],
        out_specs=pl.BlockSpec((1, c, _P), lambda i, j: (i, 0, j)),
        out_shape=jax.ShapeDtypeStruct((b, c, npix), jnp.float32),
        scratch_shapes=[pltpu.VMEM((_NE, 1), jnp.float32)],
    )(xf, embedding)
    return out.reshape(b, c, h, w)


# speedup vs baseline: 1.7468x; 1.0272x over previous
import jax
import jax.numpy as jnp
from jax.experimental import pallas as pl
from jax.experimental.pallas import tpu as pltpu

_NE = 1024
_D = 64
_P = 4096


def _body(x_ref, e_ref, o_ref, e2_ref):
    e = e_ref[...]

    @pl.when((pl.program_id(0) == 0) & (pl.program_id(1) == 0))
    def _init():
        e2_ref[...] = jnp.sum(e * e, axis=1, keepdims=True)

    x = x_ref[0]
    x2 = jnp.sum(x * x, axis=0, keepdims=True)
    e2 = e2_ref[...]
    mm2 = jax.lax.dot_general(e + e, x, (((1,), (0,)), ((), ())))
    dist = (x2 + e2) - mm2
    m = jnp.min(dist, axis=0, keepdims=True)
    jidx = jax.lax.broadcasted_iota(jnp.int32, (_NE, _P), 0).astype(jnp.float32)
    idx = jnp.min(jnp.where(dist == m, jidx, float(_NE)), axis=0,
                  keepdims=True)
    oh = (jidx == idx).astype(jnp.float32)
    q = jax.lax.dot_general(e, oh, (((0,), (0,)), ((), ())))
    o_ref[0] = x + (q - x)


def kernel(inputs, embedding):
    b, c, h, w = inputs.shape
    xf = inputs.reshape(b, c, h * w)
    npix = h * w
    out = pl.pallas_call(
        _body,
        grid=(b, npix // _P),
        in_specs=[pl.BlockSpec((1, c, _P), lambda i, j: (i, 0, j)),
                  pl.BlockSpec((_NE, _D), lambda i, j: (0, 0))],
        out_specs=pl.BlockSpec((1, c, _P), lambda i, j: (i, 0, j)),
        out_shape=jax.ShapeDtypeStruct((b, c, npix), jnp.float32),
        scratch_shapes=[pltpu.VMEM((_NE, 1), jnp.float32)],
    )(xf, embedding)
    return out.reshape(b, c, h, w)
